# baseline (speedup 1.0000x reference)
import functools
import math

import jax
import jax.numpy as jnp
from jax.experimental import pallas as pl
from jax.experimental.pallas import tpu as pltpu

D_MODEL = 128
N_HEAD = 8
NTOK = 500
NPAD = 512
SUN = 9.0
SCALE = 1.0 / math.sqrt(D_MODEL)
SCHUNK = 64


def _head_step(x_ref, wq_ref, wvf_ref, wvb_ref, wk_ref, fan_ref,
               out_ref, acc_ref):
    h = pl.program_id(0)
    xp = jnp.concatenate(
        [x_ref[0], jnp.zeros((NPAD - NTOK, D_MODEL), jnp.float32)], axis=0)
    xb = xp.astype(jnp.bfloat16)

    wq = wq_ref[:, :D_MODEL].astype(jnp.bfloat16)
    q = jax.lax.dot_general(xb, wq, (((1,), (1,)), ((), ())),
                            preferred_element_type=jnp.float32)
    k = xp * wk_ref[0]

    s_idx = jax.lax.broadcasted_iota(jnp.int32, (NPAD, 1), 0)
    kb = jnp.where(s_idx < NTOK, k, 4096.0).astype(jnp.bfloat16)
    qTb = q.astype(jnp.bfloat16).T
    s_parts = []
    for sc in range(NPAD // SCHUNK):
        ksc = kb[sc * SCHUNK:(sc + 1) * SCHUNK, :]
        acc0 = jnp.zeros((SCHUNK, NPAD), jnp.bfloat16)
        acc1 = jnp.zeros((SCHUNK, NPAD), jnp.bfloat16)
        for w in range(0, D_MODEL, 2):
            kcol = jax.lax.broadcast_in_dim(ksc[:, w], (SCHUNK, NPAD), (0,))
            qrow = jax.lax.broadcast_in_dim(qTb[w, :], (SCHUNK, NPAD), (1,))
            acc0 = acc0 + jnp.abs(kcol - qrow)
            kcol = jax.lax.broadcast_in_dim(ksc[:, w + 1], (SCHUNK, NPAD), (0,))
            qrow = jax.lax.broadcast_in_dim(qTb[w + 1, :], (SCHUNK, NPAD), (1,))
            acc1 = acc1 + jnp.abs(kcol - qrow)
        s_parts.append(acc0 + acc1)
    s_sum = jnp.concatenate(s_parts, axis=0)

    c1 = jnp.bfloat16(-SCALE * 1.4426950408889634)
    e = jnp.exp2(s_sum * c1)
    den = jnp.sum(e, axis=0, keepdims=True)
    r = (1.0 / (den.astype(jnp.float32) + 1.0)).astype(jnp.bfloat16)
    wgt = e * r

    wvf = wvf_ref[:, :D_MODEL].astype(jnp.bfloat16)
    wvb = wvb_ref[:, :D_MODEL].astype(jnp.bfloat16)
    vf = jax.lax.dot_general(xb, wvf, (((1,), (1,)), ((), ())),
                             preferred_element_type=jnp.float32
                             ).astype(jnp.bfloat16)
    vb = jax.lax.dot_general(xb, wvb, (((1,), (1,)), ((), ())),
                             preferred_element_type=jnp.float32
                             ).astype(jnp.bfloat16)

    bf = jax.lax.dot_general(wgt, vf, (((0,), (0,)), ((), ())),
                             preferred_element_type=jnp.float32)
    bb = jax.lax.dot_general(wgt, vb, (((1,), (0,)), ((), ())),
                             preferred_element_type=jnp.float32)
    bsum = bf + bb

    @pl.when(h == 0)
    def _():
        acc_ref[...] = bsum

    @pl.when(h > 0)
    def _():
        acc_ref[...] += bsum

    @pl.when(h == N_HEAD - 1)
    def _():
        g = acc_ref[...] + (SUN / 2.0)
        y = g * jax.nn.sigmoid(1.702 * g) - (SUN / 2.0)
        fan = fan_ref[:, :D_MODEL].astype(jnp.bfloat16)
        proj = jax.lax.dot_general(y.astype(jnp.bfloat16), fan,
                                   (((1,), (1,)), ((), ())),
                                   preferred_element_type=jnp.float32)
        out_ref[0] = (xp + proj)[:NTOK, :]


@jax.jit
def kernel(x, wq_w, wv_w, wk, fanout_w):
    wk3 = wk.reshape(N_HEAD, 1, D_MODEL)
    dcol = wq_w.shape[1]

    grid = (N_HEAD,)
    out = pl.pallas_call(
        _head_step,
        grid=grid,
        in_specs=[
            pl.BlockSpec((1, NTOK, D_MODEL), lambda h: (0, 0, 0)),
            pl.BlockSpec((D_MODEL, dcol), lambda h: (h, 0)),
            pl.BlockSpec((D_MODEL, dcol), lambda h: (h, 0)),
            pl.BlockSpec((D_MODEL, dcol), lambda h: (N_HEAD + h, 0)),
            pl.BlockSpec((1, 1, D_MODEL), lambda h: (h, 0, 0)),
            pl.BlockSpec((D_MODEL, dcol), lambda h: (0, 0)),
        ],
        out_specs=pl.BlockSpec((1, NTOK, D_MODEL), lambda h: (0, 0, 0)),
        out_shape=jax.ShapeDtypeStruct((1, NTOK, D_MODEL), jnp.float32),
        scratch_shapes=[pltpu.VMEM((NPAD, D_MODEL), jnp.float32)],
    )(x, wq_w, wv_w, wv_w, wk3, fanout_w)
    return out


# speedup vs baseline: 1.0177x; 1.0177x over previous
import functools
import math

import jax
import jax.numpy as jnp
from jax.experimental import pallas as pl
from jax.experimental.pallas import tpu as pltpu

D_MODEL = 128
N_HEAD = 8
NTOK = 500
NPAD = 512
SUN = 9.0
SCALE = 1.0 / math.sqrt(D_MODEL)
SCHUNK = 64


def _head_step(x_ref, wq_ref, wvf_ref, wvb_ref, wk_ref, fan_ref,
               out_ref, acc_ref):
    h = pl.program_id(0)
    xp = x_ref[...]
    xb = xp.astype(jnp.bfloat16)

    wq = wq_ref[0].astype(jnp.bfloat16)
    q = jax.lax.dot_general(xb, wq, (((1,), (1,)), ((), ())),
                            preferred_element_type=jnp.float32)
    k = xp * wk_ref[0]

    s_idx = jax.lax.broadcasted_iota(jnp.int32, (NPAD, 1), 0)
    kb = jnp.where(s_idx < NTOK, k, 4096.0).astype(jnp.bfloat16)
    qTb = q.astype(jnp.bfloat16).T
    s_parts = []
    for sc in range(NPAD // SCHUNK):
        ksc = kb[sc * SCHUNK:(sc + 1) * SCHUNK, :]
        acc0 = jnp.zeros((SCHUNK, NPAD), jnp.bfloat16)
        acc1 = jnp.zeros((SCHUNK, NPAD), jnp.bfloat16)
        for w in range(0, D_MODEL, 2):
            kcol = jax.lax.broadcast_in_dim(ksc[:, w], (SCHUNK, NPAD), (0,))
            qrow = jax.lax.broadcast_in_dim(qTb[w, :], (SCHUNK, NPAD), (1,))
            acc0 = acc0 + jnp.abs(kcol - qrow)
            kcol = jax.lax.broadcast_in_dim(ksc[:, w + 1], (SCHUNK, NPAD), (0,))
            qrow = jax.lax.broadcast_in_dim(qTb[w + 1, :], (SCHUNK, NPAD), (1,))
            acc1 = acc1 + jnp.abs(kcol - qrow)
        s_parts.append(acc0 + acc1)
    s_sum = jnp.concatenate(s_parts, axis=0)

    c1 = jnp.bfloat16(-SCALE * 1.4426950408889634)
    e = jnp.exp2(s_sum * c1)
    den = jnp.sum(e, axis=0, keepdims=True)
    r = (1.0 / (den.astype(jnp.float32) + 1.0)).astype(jnp.bfloat16)
    wgt = e * r

    wvf = wvf_ref[0].astype(jnp.bfloat16)
    wvb = wvb_ref[0].astype(jnp.bfloat16)
    vf = jax.lax.dot_general(xb, wvf, (((1,), (1,)), ((), ())),
                             preferred_element_type=jnp.float32
                             ).astype(jnp.bfloat16)
    vb = jax.lax.dot_general(xb, wvb, (((1,), (1,)), ((), ())),
                             preferred_element_type=jnp.float32
                             ).astype(jnp.bfloat16)

    bf = jax.lax.dot_general(wgt, vf, (((0,), (0,)), ((), ())),
                             preferred_element_type=jnp.float32)
    bb = jax.lax.dot_general(wgt, vb, (((1,), (0,)), ((), ())),
                             preferred_element_type=jnp.float32)
    bsum = bf + bb

    @pl.when(h == 0)
    def _():
        acc_ref[...] = bsum

    @pl.when(h > 0)
    def _():
        acc_ref[...] += bsum

    @pl.when(h == N_HEAD - 1)
    def _():
        g = acc_ref[...] + (SUN / 2.0)
        y = g * jax.nn.sigmoid(1.702 * g) - (SUN / 2.0)
        fan = fan_ref[...].astype(jnp.bfloat16)
        proj = jax.lax.dot_general(y.astype(jnp.bfloat16), fan,
                                   (((1,), (1,)), ((), ())),
                                   preferred_element_type=jnp.float32)
        out_ref[...] = xp + proj


@jax.jit
def kernel(x, wq_w, wv_w, wk, fanout_w):
    b, ntok, d = x.shape
    xp = jnp.zeros((NPAD, D_MODEL), jnp.float32).at[:NTOK].set(x[0])
    wq3 = wq_w[:, :D_MODEL].reshape(N_HEAD, D_MODEL, D_MODEL)
    wv3 = wv_w[:, :D_MODEL].reshape(2 * N_HEAD, D_MODEL, D_MODEL)
    wvf3 = wv3[:N_HEAD]
    wvb3 = wv3[N_HEAD:]
    wk3 = wk.reshape(N_HEAD, 1, D_MODEL)
    fan = fanout_w[:, :D_MODEL]

    grid = (N_HEAD,)
    out = pl.pallas_call(
        _head_step,
        grid=grid,
        in_specs=[
            pl.BlockSpec((NPAD, D_MODEL), lambda h: (0, 0)),
            pl.BlockSpec((1, D_MODEL, D_MODEL), lambda h: (h, 0, 0)),
            pl.BlockSpec((1, D_MODEL, D_MODEL), lambda h: (h, 0, 0)),
            pl.BlockSpec((1, D_MODEL, D_MODEL), lambda h: (h, 0, 0)),
            pl.BlockSpec((1, 1, D_MODEL), lambda h: (h, 0, 0)),
            pl.BlockSpec((D_MODEL, D_MODEL), lambda h: (0, 0)),
        ],
        out_specs=pl.BlockSpec((NPAD, D_MODEL), lambda h: (0, 0)),
        out_shape=jax.ShapeDtypeStruct((NPAD, D_MODEL), jnp.float32),
        scratch_shapes=[pltpu.VMEM((NPAD, D_MODEL), jnp.float32)],
    )(xp, wq3, wvf3, wvb3, wk3, fan)
    return out[None, :NTOK, :]
